# baseline (speedup 1.0000x reference)
import numpy as np

import jax
import jax.numpy as jnp
from jax.experimental import pallas as pl
from jax.experimental.pallas import tpu as pltpu

_B = 128
_N = 64
_F = 128
_L = 4
_BF = jnp.bfloat16
_F32 = jnp.float32


def _selectors():
    n = _N
    e = n * n
    i = np.repeat(np.arange(n), n)
    j = np.tile(np.arange(n), n)
    rt2 = np.zeros((e, 2 * n), dtype=np.float32)
    rt2[np.arange(e), i] += 1.0
    rt2[np.arange(e), n + j] += 1.0
    rsum = np.zeros((n, e), dtype=np.float32)
    rsum[i, np.arange(e)] = 1.0
    return jnp.asarray(rt2, _BF), jnp.asarray(rsum, _BF)


def _mm(a, b):
    return jax.lax.dot_general(a, b, (((a.ndim - 1,), (0,)), ((), ())),
                               preferred_element_type=_F32)


def _egnn_body(z_ref, at_ref, injb_ref, Wia_ref, Wiz_ref,
               W0a_ref, W0b_ref, b0_ref, W0C_ref,
               W1_ref, b1_ref,
               Wnh_ref, Wna_ref, nb0_ref, nW1_ref, nb1_ref,
               Wc_ref, bc_ref, W1B_ref,
               RT2_ref, Rsum_ref,
               out_ref):
    n, f = _N, _F
    h = (_mm(at_ref[...].astype(_BF), Wia_ref[...])
         + _mm(z_ref[0].astype(_BF), Wiz_ref[...]) + injb_ref[...])
    coords = jnp.zeros((n, f), dtype=_F32)
    RT2 = RT2_ref[...]
    Rsum = Rsum_ref[...]

    for l in range(_L):
        cc = jnp.concatenate([coords, -coords], axis=0).astype(_BF)
        rel16 = _mm(RT2, cc).astype(_BF)

        h16 = h.astype(_BF)
        A = _mm(h16, W0a_ref[l])
        Bc = _mm(h16, W0b_ref[l]) + b0_ref[l][None, :]
        AB = jnp.concatenate([A, Bc], axis=0).astype(_BF)
        pre2 = _mm(RT2, AB) + _mm(rel16 * rel16, W0C_ref[l])
        r16 = jnp.maximum(pre2, 0.0).astype(_BF)

        S = _mm(Rsum, r16) - jnp.maximum(A + Bc, 0.0)
        agg = _mm(S.astype(_BF), W1_ref[l]) + float(n - 1) * b1_ref[l][None, :]
        hid = jnp.maximum(_mm(h16, Wnh_ref[l])
                          + _mm(agg.astype(_BF), Wna_ref[l])
                          + nb0_ref[l][None, :], 0.0)
        hn = _mm(hid.astype(_BF), nW1_ref[l]) + nb1_ref[l][None, :]

        t16 = jnp.maximum(_mm(r16, Wc_ref[l]) + bc_ref[l][None, :],
                          0.0).astype(_BF)
        cwb = _mm(t16, W1B_ref[l]).astype(_BF)
        coords = coords + _mm(Rsum, cwb * rel16)
        h = hn

    out_ref[0] = coords


def kernel(z, atom_types, inj_W, inj_b, edge_W0, edge_b0, edge_W1, edge_b1,
           node_W0, node_b0, node_W1, node_b1, coord_W0, coord_b0, coord_W1):
    f = _F
    Wia = inj_W[:f].astype(_BF)
    Wiz = inj_W[f:].astype(_BF)
    W0a = edge_W0[:, :f, :].astype(_BF)
    W0b = edge_W0[:, f:2 * f, :].astype(_BF)
    W0C = jnp.broadcast_to(edge_W0[:, 2 * f, :][:, None, :],
                           (_L, f, f)).astype(_BF)
    Wnh = node_W0[:, :f, :].astype(_BF)
    Wna = node_W0[:, f:, :].astype(_BF)
    Wc = jnp.einsum("lij,ljk->lik", edge_W1, coord_W0).astype(_BF)
    bc = jnp.einsum("lj,ljk->lk", edge_b1, coord_W0) + coord_b0
    W1B = jnp.broadcast_to(coord_W1, (_L, f, f)).astype(_BF)
    W1 = edge_W1.astype(_BF)
    nW1 = node_W1.astype(_BF)
    RT2, Rsum = _selectors()

    full = lambda a: pl.BlockSpec(a.shape, lambda b: (0,) * a.ndim)
    injb2 = inj_b.reshape(1, f)
    z3 = z.reshape(_B, 1, z.shape[1])

    out = pl.pallas_call(
        _egnn_body,
        grid=(_B,),
        in_specs=[
            pl.BlockSpec((1, 1, z.shape[1]), lambda b: (b, 0, 0)),
            pl.BlockSpec((_N, f), lambda b: (b, 0)),
            full(injb2), full(Wia), full(Wiz),
            full(W0a), full(W0b), full(edge_b0), full(W0C),
            full(W1), full(edge_b1),
            full(Wnh), full(Wna), full(node_b0), full(nW1), full(node_b1),
            full(Wc), full(bc), full(W1B),
            full(RT2), full(Rsum),
        ],
        out_specs=pl.BlockSpec((1, _N, f), lambda b: (b, 0, 0)),
        out_shape=jax.ShapeDtypeStruct((_B, _N, f), jnp.float32),
        compiler_params=pltpu.CompilerParams(
            dimension_semantics=("arbitrary",),
        ),
    )(z3, atom_types, injb2, Wia, Wiz, W0a, W0b, edge_b0, W0C,
      W1, edge_b1, Wnh, Wna, node_b0, nW1, node_b1, Wc, bc, W1B,
      RT2, Rsum)
    return out[:, :, :3]


# speedup vs baseline: 1.6237x; 1.6237x over previous
import numpy as np

import jax
import jax.numpy as jnp
from jax.experimental import pallas as pl
from jax.experimental.pallas import tpu as pltpu

_B = 128
_N = 64
_F = 128
_L = 4
_MPB = 2
_BF = jnp.bfloat16
_F32 = jnp.float32


def _rsum():
    n = _N
    e = n * n
    i = np.repeat(np.arange(n), n)
    rsum = np.zeros((n, e), dtype=np.float32)
    rsum[i, np.arange(e)] = 1.0
    return jnp.asarray(rsum, _BF)


def _mm(a, b):
    return jax.lax.dot_general(a, b, (((a.ndim - 1,), (0,)), ((), ())),
                               preferred_element_type=_F32)


def _layer(h, coords, l, refs):
    (W0a_ref, W0b_ref, b0_ref, w0c_ref, W1_ref, b1_ref,
     Wnh_ref, Wna_ref, nb0_ref, nW1_ref, nb1_ref,
     Wc_ref, bc_ref, w1c_ref, Rsum) = refs
    n, f = _N, _F
    rel3 = coords[:, None, :] - coords[None, :, :]
    dsq3 = jnp.sum(rel3 * rel3, axis=2, keepdims=True)

    h16 = h.astype(_BF)
    A = _mm(h16, W0a_ref[l])
    Bc = _mm(h16, W0b_ref[l]) + b0_ref[l][None, :]
    pre3 = (A[:, None, :] + Bc[None, :, :]
            + dsq3 * w0c_ref[l][None, None, :])
    r16 = jnp.maximum(pre3, 0.0).astype(_BF).reshape(n * n, f)

    S = _mm(Rsum, r16) - jnp.maximum(A + Bc, 0.0)
    agg = _mm(S.astype(_BF), W1_ref[l]) + float(n - 1) * b1_ref[l][None, :]
    hid = jnp.maximum(_mm(h16, Wnh_ref[l])
                      + _mm(agg.astype(_BF), Wna_ref[l])
                      + nb0_ref[l][None, :], 0.0)
    hn = _mm(hid.astype(_BF), nW1_ref[l]) + nb1_ref[l][None, :]

    t = jnp.maximum(_mm(r16, Wc_ref[l]) + bc_ref[l][None, :], 0.0)
    cw = jnp.sum(t * w1c_ref[l][None, :], axis=1, keepdims=True)
    prod16 = (cw * rel3.reshape(n * n, f)).astype(_BF)
    coords = coords + _mm(Rsum, prod16)
    return hn, coords


def _egnn_body(z_ref, at_ref, injb_ref, Wia_ref, Wiz_ref,
               W0a_ref, W0b_ref, b0_ref, w0c_ref,
               W1_ref, b1_ref,
               Wnh_ref, Wna_ref, nb0_ref, nW1_ref, nb1_ref,
               Wc_ref, bc_ref, w1c_ref,
               Rsum_ref,
               out_ref):
    n, f = _N, _F
    refs = (W0a_ref, W0b_ref, b0_ref, w0c_ref, W1_ref, b1_ref,
            Wnh_ref, Wna_ref, nb0_ref, nW1_ref, nb1_ref,
            Wc_ref, bc_ref, w1c_ref, Rsum_ref[...])
    hs, cs = [], []
    for m in range(_MPB):
        at = at_ref[m * n:(m + 1) * n, :]
        zrow = z_ref[0, m:m + 1, :]
        hs.append(_mm(at.astype(_BF), Wia_ref[...])
                  + _mm(zrow.astype(_BF), Wiz_ref[...]) + injb_ref[...])
        cs.append(jnp.zeros((n, f), dtype=_F32))
    for l in range(_L):
        for m in range(_MPB):
            hs[m], cs[m] = _layer(hs[m], cs[m], l, refs)
    for m in range(_MPB):
        out_ref[m] = cs[m]


def kernel(z, atom_types, inj_W, inj_b, edge_W0, edge_b0, edge_W1, edge_b1,
           node_W0, node_b0, node_W1, node_b1, coord_W0, coord_b0, coord_W1):
    f = _F
    Wia = inj_W[:f].astype(_BF)
    Wiz = inj_W[f:].astype(_BF)
    W0a = edge_W0[:, :f, :].astype(_BF)
    W0b = edge_W0[:, f:2 * f, :].astype(_BF)
    w0c = edge_W0[:, 2 * f, :]
    Wnh = node_W0[:, :f, :].astype(_BF)
    Wna = node_W0[:, f:, :].astype(_BF)
    Wc = jnp.einsum("lij,ljk->lik", edge_W1, coord_W0).astype(_BF)
    bc = jnp.einsum("lj,ljk->lk", edge_b1, coord_W0) + coord_b0
    w1c = coord_W1[:, :, 0]
    W1 = edge_W1.astype(_BF)
    nW1 = node_W1.astype(_BF)
    Rsum = _rsum()

    full = lambda a: pl.BlockSpec(a.shape, lambda b: (0,) * a.ndim)
    injb2 = inj_b.reshape(1, f)
    z3 = z.reshape(_B // _MPB, _MPB, z.shape[1])

    out = pl.pallas_call(
        _egnn_body,
        grid=(_B // _MPB,),
        in_specs=[
            pl.BlockSpec((1, _MPB, z.shape[1]), lambda b: (b, 0, 0)),
            pl.BlockSpec((_MPB * _N, f), lambda b: (b, 0)),
            full(injb2), full(Wia), full(Wiz),
            full(W0a), full(W0b), full(edge_b0), full(w0c),
            full(W1), full(edge_b1),
            full(Wnh), full(Wna), full(node_b0), full(nW1), full(node_b1),
            full(Wc), full(bc), full(w1c),
            full(Rsum),
        ],
        out_specs=pl.BlockSpec((_MPB, _N, f), lambda b: (b, 0, 0)),
        out_shape=jax.ShapeDtypeStruct((_B, _N, f), jnp.float32),
        compiler_params=pltpu.CompilerParams(
            dimension_semantics=("arbitrary",),
        ),
    )(z3, atom_types, injb2, Wia, Wiz, W0a, W0b, edge_b0, w0c,
      W1, edge_b1, Wnh, Wna, node_b0, nW1, node_b1, Wc, bc, w1c,
      Rsum)
    return out[:, :, :3]


# speedup vs baseline: 1.8765x; 1.1557x over previous
import numpy as np

import jax
import jax.numpy as jnp
from jax.experimental import pallas as pl
from jax.experimental.pallas import tpu as pltpu

_B = 128
_N = 64
_F = 128
_L = 4
_MPB = 2
_BF = jnp.bfloat16
_F32 = jnp.float32


def _selectors():
    n = _N
    e = n * n
    i = np.repeat(np.arange(n), n)
    j = np.tile(np.arange(n), n)
    rsum = np.zeros((n, e), dtype=np.float32)
    rsum[i, np.arange(e)] = 1.0
    rt2 = np.zeros((e, 2 * n), dtype=np.float32)
    rt2[np.arange(e), i] += 1.0
    rt2[np.arange(e), n + j] += 1.0
    return jnp.asarray(rsum, _BF), jnp.asarray(rt2, _BF)


def _mm(a, b):
    return jax.lax.dot_general(a, b, (((a.ndim - 1,), (0,)), ((), ())),
                               preferred_element_type=_F32)


def _egnn_body(z_ref, at_ref, injb_ref, Wia_ref, Wiz_ref,
               W0a_ref, W0b_ref, b0_ref, W0C_ref,
               W1_ref, b1_ref,
               Wnh_ref, Wna_ref, nb0_ref, nW1_ref, nb1_ref,
               Wc_ref, bc_ref, w1c_ref,
               Rsum_ref, RT2_ref,
               out_ref):
    n, f, m = _N, _F, _MPB
    Rsum = Rsum_ref[...]
    RT2 = RT2_ref[...]

    z_exp = jnp.broadcast_to(z_ref[0][:, None, :], (m, n, f)).reshape(m * n, f)
    h = (_mm(at_ref[...].astype(_BF), Wia_ref[...])
         + _mm(z_exp.astype(_BF), Wiz_ref[...]) + injb_ref[...])
    cs = [jnp.zeros((n, f), dtype=_F32) for _ in range(m)]

    for l in range(_L):
        h16 = h.astype(_BF)
        A = _mm(h16, W0a_ref[l])
        Bc = _mm(h16, W0b_ref[l]) + b0_ref[l][None, :]

        rel3s = [cs[k][:, None, :] - cs[k][None, :, :] for k in range(m)]
        sq16s = [(r * r).reshape(n * n, f).astype(_BF) for r in rel3s]
        dsqws = [_mm(s, W0C_ref[l]).reshape(n, n, f) for s in sq16s]
        r16s, Ss = [], []
        for k in range(m):
            Ak = A[k * n:(k + 1) * n, :]
            Bk = Bc[k * n:(k + 1) * n, :]
            pre3 = Ak[:, None, :] + Bk[None, :, :] + dsqws[k]
            r16s.append(jnp.maximum(pre3, 0.0).astype(_BF).reshape(n * n, f))
        for k in range(m):
            Ss.append(_mm(Rsum, r16s[k]))

        S = jnp.concatenate(Ss, axis=0) - jnp.maximum(A + Bc, 0.0)
        agg = _mm(S.astype(_BF), W1_ref[l]) + float(n - 1) * b1_ref[l][None, :]
        hid = jnp.maximum(_mm(h16, Wnh_ref[l])
                          + _mm(agg.astype(_BF), Wna_ref[l])
                          + nb0_ref[l][None, :], 0.0)
        h = _mm(hid.astype(_BF), nW1_ref[l]) + nb1_ref[l][None, :]

        ts = [jnp.maximum(_mm(r16s[k], Wc_ref[l]) + bc_ref[l][None, :], 0.0)
              for k in range(m)]
        cws = [jnp.sum(t * w1c_ref[l][None, :], axis=1, keepdims=True)
               for t in ts]
        for k in range(m):
            prod16 = (cws[k] * rel3s[k].reshape(n * n, f)).astype(_BF)
            cs[k] = cs[k] + _mm(Rsum, prod16)

    for k in range(m):
        out_ref[k] = cs[k]


def kernel(z, atom_types, inj_W, inj_b, edge_W0, edge_b0, edge_W1, edge_b1,
           node_W0, node_b0, node_W1, node_b1, coord_W0, coord_b0, coord_W1):
    f = _F
    Wia = inj_W[:f].astype(_BF)
    Wiz = inj_W[f:].astype(_BF)
    W0a = edge_W0[:, :f, :].astype(_BF)
    W0b = edge_W0[:, f:2 * f, :].astype(_BF)
    W0C = jnp.broadcast_to(edge_W0[:, 2 * f, :][:, None, :],
                           (_L, f, f)).astype(_BF)
    Wnh = node_W0[:, :f, :].astype(_BF)
    Wna = node_W0[:, f:, :].astype(_BF)
    Wc = jnp.einsum("lij,ljk->lik", edge_W1, coord_W0).astype(_BF)
    bc = jnp.einsum("lj,ljk->lk", edge_b1, coord_W0) + coord_b0
    w1c = coord_W1[:, :, 0]
    W1 = edge_W1.astype(_BF)
    nW1 = node_W1.astype(_BF)
    Rsum, RT2 = _selectors()

    full = lambda a: pl.BlockSpec(a.shape, lambda b: (0,) * a.ndim)
    injb2 = inj_b.reshape(1, f)
    z3 = z.reshape(_B // _MPB, _MPB, z.shape[1])

    out = pl.pallas_call(
        _egnn_body,
        grid=(_B // _MPB,),
        in_specs=[
            pl.BlockSpec((1, _MPB, z.shape[1]), lambda b: (b, 0, 0)),
            pl.BlockSpec((_MPB * _N, f), lambda b: (b, 0)),
            full(injb2), full(Wia), full(Wiz),
            full(W0a), full(W0b), full(edge_b0), full(W0C),
            full(W1), full(edge_b1),
            full(Wnh), full(Wna), full(node_b0), full(nW1), full(node_b1),
            full(Wc), full(bc), full(w1c),
            full(Rsum), full(RT2),
        ],
        out_specs=pl.BlockSpec((_MPB, _N, f), lambda b: (b, 0, 0)),
        out_shape=jax.ShapeDtypeStruct((_B, _N, f), jnp.float32),
        compiler_params=pltpu.CompilerParams(
            dimension_semantics=("arbitrary",),
        ),
    )(z3, atom_types, injb2, Wia, Wiz, W0a, W0b, edge_b0, W0C,
      W1, edge_b1, Wnh, Wna, node_b0, nW1, node_b1, Wc, bc, w1c,
      Rsum, RT2)
    return out[:, :, :3]


# speedup vs baseline: 2.1085x; 1.1236x over previous
import numpy as np

import jax
import jax.numpy as jnp
from jax.experimental import pallas as pl
from jax.experimental.pallas import tpu as pltpu

_B = 128
_N = 64
_F = 128
_L = 4
_MPB = 4
_BF = jnp.bfloat16
_F32 = jnp.float32


def _selectors():
    n = _N
    e = n * n
    i = np.repeat(np.arange(n), n)
    j = np.tile(np.arange(n), n)
    rsum = np.zeros((n, e), dtype=np.float32)
    rsum[i, np.arange(e)] = 1.0
    rt2 = np.zeros((e, 2 * n), dtype=np.float32)
    rt2[np.arange(e), i] += 1.0
    rt2[np.arange(e), n + j] += 1.0
    return jnp.asarray(rsum, _BF), jnp.asarray(rt2, _BF)


def _mm(a, b):
    return jax.lax.dot_general(a, b, (((a.ndim - 1,), (0,)), ((), ())),
                               preferred_element_type=_F32)


def _egnn_body(z_ref, at_ref, injb_ref, Wia_ref, Wiz_ref,
               W0a_ref, W0b_ref, b0_ref, W0C_ref,
               W1_ref, b1_ref,
               Wnh_ref, Wna_ref, nb0_ref, nW1_ref, nb1_ref,
               Wc_ref, bc_ref, w1c_ref,
               Rsum_ref, RT2_ref,
               out_ref):
    n, f, m = _N, _F, _MPB
    Rsum = Rsum_ref[...]
    RT2 = RT2_ref[...]

    z_exp = jnp.broadcast_to(z_ref[0][:, None, :], (m, n, f)).reshape(m * n, f)
    h = (_mm(at_ref[...].astype(_BF), Wia_ref[...])
         + _mm(z_exp.astype(_BF), Wiz_ref[...]) + injb_ref[...])
    cs = [jnp.zeros((n, f), dtype=_F32) for _ in range(m)]

    for l in range(_L):
        h16 = h.astype(_BF)
        A = _mm(h16, W0a_ref[l])
        Bc = _mm(h16, W0b_ref[l]) + b0_ref[l][None, :]

        rel3s = [cs[k][:, None, :] - cs[k][None, :, :] for k in range(m)]
        sq16s = [(r * r).reshape(n * n, f).astype(_BF) for r in rel3s]
        dsqws = [_mm(s, W0C_ref[l]).reshape(n, n, f) for s in sq16s]
        r16s, Ss = [], []
        for k in range(m):
            Ak = A[k * n:(k + 1) * n, :]
            Bk = Bc[k * n:(k + 1) * n, :]
            pre3 = Ak[:, None, :] + Bk[None, :, :] + dsqws[k]
            r16s.append(jnp.maximum(pre3, 0.0).astype(_BF).reshape(n * n, f))
        for k in range(m):
            Ss.append(_mm(Rsum, r16s[k]))

        S = jnp.concatenate(Ss, axis=0) - jnp.maximum(A + Bc, 0.0)
        agg = _mm(S.astype(_BF), W1_ref[l]) + float(n - 1) * b1_ref[l][None, :]
        hid = jnp.maximum(_mm(h16, Wnh_ref[l])
                          + _mm(agg.astype(_BF), Wna_ref[l])
                          + nb0_ref[l][None, :], 0.0)
        h = _mm(hid.astype(_BF), nW1_ref[l]) + nb1_ref[l][None, :]

        ts = [jnp.maximum(_mm(r16s[k], Wc_ref[l]) + bc_ref[l][None, :], 0.0)
              for k in range(m)]
        cws = [jnp.sum(t * w1c_ref[l][None, :], axis=1, keepdims=True)
               for t in ts]
        for k in range(m):
            prod16 = (cws[k] * rel3s[k].reshape(n * n, f)).astype(_BF)
            cs[k] = cs[k] + _mm(Rsum, prod16)

    for k in range(m):
        out_ref[k] = cs[k]


def kernel(z, atom_types, inj_W, inj_b, edge_W0, edge_b0, edge_W1, edge_b1,
           node_W0, node_b0, node_W1, node_b1, coord_W0, coord_b0, coord_W1):
    f = _F
    Wia = inj_W[:f].astype(_BF)
    Wiz = inj_W[f:].astype(_BF)
    W0a = edge_W0[:, :f, :].astype(_BF)
    W0b = edge_W0[:, f:2 * f, :].astype(_BF)
    W0C = jnp.broadcast_to(edge_W0[:, 2 * f, :][:, None, :],
                           (_L, f, f)).astype(_BF)
    Wnh = node_W0[:, :f, :].astype(_BF)
    Wna = node_W0[:, f:, :].astype(_BF)
    Wc = jnp.einsum("lij,ljk->lik", edge_W1, coord_W0).astype(_BF)
    bc = jnp.einsum("lj,ljk->lk", edge_b1, coord_W0) + coord_b0
    w1c = coord_W1[:, :, 0]
    W1 = edge_W1.astype(_BF)
    nW1 = node_W1.astype(_BF)
    Rsum, RT2 = _selectors()

    full = lambda a: pl.BlockSpec(a.shape, lambda b: (0,) * a.ndim)
    injb2 = inj_b.reshape(1, f)
    z3 = z.reshape(_B // _MPB, _MPB, z.shape[1])

    out = pl.pallas_call(
        _egnn_body,
        grid=(_B // _MPB,),
        in_specs=[
            pl.BlockSpec((1, _MPB, z.shape[1]), lambda b: (b, 0, 0)),
            pl.BlockSpec((_MPB * _N, f), lambda b: (b, 0)),
            full(injb2), full(Wia), full(Wiz),
            full(W0a), full(W0b), full(edge_b0), full(W0C),
            full(W1), full(edge_b1),
            full(Wnh), full(Wna), full(node_b0), full(nW1), full(node_b1),
            full(Wc), full(bc), full(w1c),
            full(Rsum), full(RT2),
        ],
        out_specs=pl.BlockSpec((_MPB, _N, f), lambda b: (b, 0, 0)),
        out_shape=jax.ShapeDtypeStruct((_B, _N, f), jnp.float32),
        compiler_params=pltpu.CompilerParams(
            dimension_semantics=("arbitrary",),
        ),
    )(z3, atom_types, injb2, Wia, Wiz, W0a, W0b, edge_b0, W0C,
      W1, edge_b1, Wnh, Wna, node_b0, nW1, node_b1, Wc, bc, w1c,
      Rsum, RT2)
    return out[:, :, :3]
